# Initial kernel scaffold; baseline (speedup 1.0000x reference)
import jax
import jax.numpy as jnp
from jax.experimental import pallas as pl


def kernel(x, edge_index, W1, b1, Wmu, bmu, Wls, bls):
    raise NotImplementedError("write your pallas kernel here")


# speedup vs baseline: 9.6911x; 9.6911x over previous
import functools

import jax
import jax.numpy as jnp
from jax import lax
from jax.experimental import pallas as pl
from jax.experimental.pallas import tpu as pltpu
from jax.experimental.pallas import tpu_sc as plsc

N_NODES = 10000
N_EDGES = 320000
IN_CH = 128
OUT_CH = 48
HID = 2 * OUT_CH

CHUNK = 128
N_CHUNKS = N_EDGES // CHUNK
NC = 2
NS = 16
N_PAD = 10240
ROWS_PER_SUB = N_PAD // NS
CW = 128
PIECE = 64
N_PIECES = ROWS_PER_SUB // PIECE
NJ = HID // 16
DCOL = HID
CHUNKS_PER_SUB = N_CHUNKS // NS
CHUNKS_SUB_REM = N_CHUNKS - CHUNKS_PER_SUB * NS

_mesh = plsc.VectorSubcoreMesh(core_axis_name="c", subcore_axis_name="s")


def _rsqrt16(d):
    i = lax.bitcast_convert_type(d, jnp.int32)
    i = jnp.int32(0x5F3759DF) - lax.shift_right_arithmetic(i, 1)
    y = lax.bitcast_convert_type(i, jnp.float32)
    for _ in range(3):
        y = y * (1.5 - 0.5 * d * y * y)
    return y


@functools.partial(
    pl.kernel,
    out_type=(
        jax.ShapeDtypeStruct((N_PAD, CW), jnp.float32),
        jax.ShapeDtypeStruct((NC * N_PAD, CW), jnp.float32),
    ),
    mesh=_mesh,
    scratch_types=[
        pltpu.VMEM((CHUNK,), jnp.int32),
        pltpu.VMEM((CHUNK,), jnp.int32),
        pltpu.VMEM((CHUNK, CW), jnp.float32),
        pltpu.VMEM((PIECE, CW), jnp.float32),
        pltpu.VMEM((PIECE, CW), jnp.float32),
        pltpu.VMEM((CW,), jnp.float32),
        pltpu.VMEM((PIECE, CW), jnp.float32),
        pltpu.VMEM_SHARED((N_PAD, CW), jnp.float32),
        pltpu.SemaphoreType.DMA,
    ],
)
def _gcn_sc_kernel(h1_hbm, src_hbm, dst_hbm, b1_hbm, g_hbm, ptab_hbm,
                   src_v, dst_v, rows_v, a_v, b_v, bias_v, zero_v,
                   acc_sh, sem):
    cid = lax.axis_index("c")
    sid = lax.axis_index("s")
    base = sid * ROWS_PER_SUB
    roff = cid * N_PAD
    zrow = jnp.zeros((16,), jnp.float32)
    onerow = jnp.ones((16,), jnp.float32)

    extra = jnp.minimum(sid, CHUNKS_SUB_REM)
    cstart = sid * CHUNKS_PER_SUB + extra
    cnum = CHUNKS_PER_SUB + jnp.where(sid < CHUNKS_SUB_REM, 1, 0)

    def fill_body(r, carry):
        for j in range(CW // 16):
            rows_v[r, pl.ds(j * 16, 16)] = onerow
        return carry

    lax.fori_loop(0, CHUNK, fill_body, 0)

    def zfill_body(r, carry):
        for j in range(CW // 16):
            zero_v[r, pl.ds(j * 16, 16)] = zrow
        return carry

    lax.fori_loop(0, PIECE, zfill_body, 0)

    def zero_slab(k, carry):
        pltpu.sync_copy(zero_v, acc_sh.at[pl.ds(base + k * PIECE, PIECE)])
        return carry

    lax.fori_loop(0, N_PIECES, zero_slab, 0)
    pltpu.sync_copy(b1_hbm, bias_v)
    plsc.subcore_barrier()

    def hist_body(i, carry):
        pltpu.sync_copy(dst_hbm.at[cstart + i], dst_v)
        pltpu.sync_copy(rows_v, acc_sh.at[dst_v], add=True)
        return carry

    lax.fori_loop(0, cnum, hist_body, 0)
    plsc.subcore_barrier()

    def p1_piece(k, carry):
        pltpu.sync_copy(h1_hbm.at[pl.ds(base + k * PIECE, PIECE)], a_v)
        pltpu.sync_copy(acc_sh.at[pl.ds(base + k * PIECE, PIECE)], b_v)

        def p1_row(r, c2):
            y = _rsqrt16(b_v[r, pl.ds(0, 16)] + 1.0)
            for j in range(NJ):
                a_v[r, pl.ds(j * 16, 16)] = a_v[r, pl.ds(j * 16, 16)] * y
            a_v[r, pl.ds(DCOL, 16)] = y
            return c2

        lax.fori_loop(0, PIECE, p1_row, 0)
        pltpu.sync_copy(a_v, ptab_hbm.at[pl.ds(roff + base + k * PIECE, PIECE)])
        pltpu.sync_copy(zero_v, acc_sh.at[pl.ds(base + k * PIECE, PIECE)])
        return carry

    lax.fori_loop(0, N_PIECES, p1_piece, 0)
    plsc.subcore_barrier()

    def agg_body(i, carry):
        pltpu.sync_copy(src_hbm.at[cstart + i], src_v)
        pltpu.sync_copy(dst_hbm.at[cstart + i], dst_v)

        def adj(t, c2):
            src_v[pl.ds(t * 16, 16)] = src_v[pl.ds(t * 16, 16)] + roff
            return c2

        lax.fori_loop(0, CHUNK // 16, adj, 0)
        pltpu.async_copy(ptab_hbm.at[src_v], rows_v, sem).wait()
        pltpu.sync_copy(rows_v, acc_sh.at[dst_v], add=True)
        return carry

    lax.fori_loop(0, cnum, agg_body, 0)
    plsc.subcore_barrier()

    def mid_piece(k, carry):
        pltpu.sync_copy(acc_sh.at[pl.ds(base + k * PIECE, PIECE)], a_v)
        pltpu.sync_copy(ptab_hbm.at[pl.ds(roff + base + k * PIECE, PIECE)], b_v)

        def mid_row(r, c2):
            y = b_v[r, pl.ds(DCOL, 16)]
            for j in range(NJ):
                s = a_v[r, pl.ds(j * 16, 16)] + b_v[r, pl.ds(j * 16, 16)]
                h = jnp.maximum(s * y + bias_v[pl.ds(j * 16, 16)], 0.0)
                b_v[r, pl.ds(j * 16, 16)] = h * y
            return c2

        lax.fori_loop(0, PIECE, mid_row, 0)
        pltpu.sync_copy(b_v, ptab_hbm.at[pl.ds(roff + base + k * PIECE, PIECE)])
        pltpu.sync_copy(zero_v, acc_sh.at[pl.ds(base + k * PIECE, PIECE)])
        return carry

    lax.fori_loop(0, N_PIECES, mid_piece, 0)
    plsc.subcore_barrier()

    lax.fori_loop(0, cnum, agg_body, 0)
    plsc.subcore_barrier()

    @pl.when(cid == 0)
    def _():
        def out_piece(k, carry):
            pltpu.sync_copy(acc_sh.at[pl.ds(base + k * PIECE, PIECE)], a_v)
            pltpu.sync_copy(
                ptab_hbm.at[pl.ds(roff + base + k * PIECE, PIECE)], b_v)

            def out_row(r, c2):
                y = b_v[r, pl.ds(DCOL, 16)]
                for j in range(NJ):
                    s = a_v[r, pl.ds(j * 16, 16)] + b_v[r, pl.ds(j * 16, 16)]
                    a_v[r, pl.ds(j * 16, 16)] = s * y
                return c2

            lax.fori_loop(0, PIECE, out_row, 0)
            pltpu.sync_copy(a_v, g_hbm.at[pl.ds(base + k * PIECE, PIECE)])
            return carry

        lax.fori_loop(0, N_PIECES, out_piece, 0)


def _tc_pre(x_ref, w1_ref, h1_ref):
    h1 = jnp.dot(x_ref[...], w1_ref[...], preferred_element_type=jnp.float32)
    h1_ref[:N_NODES, :] = h1
    h1_ref[N_NODES:, :] = jnp.zeros((N_PAD - N_NODES, CW), jnp.float32)


def _tc_post(g_ref, wcat_ref, bcat_ref, out_ref):
    g = g_ref[:N_NODES, :]
    out_ref[...] = (
        jnp.dot(g, wcat_ref[...], preferred_element_type=jnp.float32)
        + bcat_ref[...]
    )


def kernel(x, edge_index, W1, b1, Wmu, bmu, Wls, bls):
    ei = edge_index.astype(jnp.int32)
    src = ei[0].reshape(N_CHUNKS, CHUNK)
    dst = ei[1].reshape(N_CHUNKS, CHUNK)

    w1p = jnp.pad(W1, ((0, 0), (0, CW - HID)))
    h1 = pl.pallas_call(
        _tc_pre,
        out_shape=jax.ShapeDtypeStruct((N_PAD, CW), jnp.float32),
    )(x, w1p)

    b1p = jnp.pad(b1, (0, CW - HID))
    g, _ = _gcn_sc_kernel(h1, src, dst, b1p)

    wcat = jnp.pad(jnp.concatenate([Wmu, Wls], axis=1), ((0, CW - HID), (0, 0)))
    bcat = jnp.concatenate([bmu, bls]).reshape(1, 2 * OUT_CH)
    out = pl.pallas_call(
        _tc_post,
        out_shape=jax.ShapeDtypeStruct((N_NODES, 2 * OUT_CH), jnp.float32),
    )(g, wcat, bcat)

    return out[:, :OUT_CH], out[:, OUT_CH:]
